# baseline (speedup 1.0000x reference)
import functools

import jax
import jax.numpy as jnp
from jax import lax
from jax.experimental import pallas as pl
from jax.experimental.pallas import tpu as pltpu
from jax.experimental.pallas import tpu_sc as plsc

VERTEXES = 100000
EDGES = 1600000

NUM_CORES = 2
NUM_SUBCORES = 16
NUM_TILES = NUM_CORES * NUM_SUBCORES

V_SLICE = 6256
V_PAD = V_SLICE * NUM_SUBCORES

CHUNK = 10000
EDGES_PER_TILE = EDGES // NUM_TILES
TILE_CHUNKS = EDGES_PER_TILE // CHUNK

_LANES = 16

_MESH = plsc.VectorSubcoreMesh(core_axis_name="c", subcore_axis_name="s",
                               num_cores=NUM_CORES, num_subcores=NUM_SUBCORES)


@functools.partial(
    pl.kernel,
    out_type=(jax.ShapeDtypeStruct((NUM_CORES * 2 * V_PAD,), jnp.float32),
              jax.ShapeDtypeStruct((EDGES,), jnp.int32),
              jax.ShapeDtypeStruct((EDGES,), jnp.int32)),
    mesh=_MESH,
    scratch_types=(
        pltpu.VMEM_SHARED((V_PAD,), jnp.float32),
        pltpu.VMEM_SHARED((V_PAD,), jnp.float32),
        pltpu.VMEM((CHUNK, 3), jnp.int32),
        pltpu.VMEM((CHUNK,), jnp.int32),
        pltpu.VMEM((CHUNK,), jnp.float32),
        pltpu.VMEM((V_SLICE,), jnp.float32),
    ),
    compiler_params=pltpu.CompilerParams(use_tc_tiling_on_sc=False,
                                         needs_layout_passes=False),
)
def _count_partials(x_hbm, part_hbm, recv_out, send_out, cnt_fwd, cnt_bwd,
                    xrow_buf, idx_buf, ones_buf, zero_buf):
    c = lax.axis_index("c")
    s = lax.axis_index("s")
    lane_iota = lax.iota(jnp.int32, _LANES)

    def _fill(i, _):
        ones_buf[pl.ds(i * _LANES, _LANES)] = jnp.full((_LANES,), 1.0, jnp.float32)
        return 0
    lax.fori_loop(0, CHUNK // _LANES, _fill, 0)

    def _zero(i, _):
        zero_buf[pl.ds(i * _LANES, _LANES)] = jnp.zeros((_LANES,), jnp.float32)
        return 0
    lax.fori_loop(0, V_SLICE // _LANES, _zero, 0)
    voff = s * V_SLICE
    pltpu.sync_copy(zero_buf, cnt_fwd.at[pl.ds(voff, V_SLICE)])
    pltpu.sync_copy(zero_buf, cnt_bwd.at[pl.ds(voff, V_SLICE)])
    plsc.subcore_barrier()

    gbase = (s * NUM_CORES + c) * EDGES_PER_TILE

    def _deinterleave(col):
        def _d(j, _):
            rows = lane_iota + j * _LANES
            cols = jnp.full((_LANES,), col, jnp.int32)
            idx_buf[pl.ds(j * _LANES, _LANES)] = plsc.load_gather(
                xrow_buf, [rows, cols])
            return 0
        lax.fori_loop(0, CHUNK // _LANES, _d, 0)

    def _hist(k, _):
        base = gbase + k * CHUNK
        pltpu.sync_copy(x_hbm.at[pl.ds(base, CHUNK), :], xrow_buf)
        _deinterleave(2)
        pltpu.sync_copy(ones_buf, cnt_fwd.at[idx_buf], add=True)
        pltpu.sync_copy(idx_buf, recv_out.at[pl.ds(base, CHUNK)])
        _deinterleave(0)
        pltpu.sync_copy(ones_buf, cnt_bwd.at[idx_buf], add=True)
        pltpu.sync_copy(idx_buf, send_out.at[pl.ds(base, CHUNK)])
        return 0
    lax.fori_loop(0, TILE_CHUNKS, _hist, 0)
    plsc.subcore_barrier()

    pltpu.sync_copy(cnt_fwd.at[pl.ds(voff, V_SLICE)], zero_buf)
    pltpu.sync_copy(zero_buf, part_hbm.at[pl.ds(c * 2 * V_PAD + voff, V_SLICE)])
    pltpu.sync_copy(cnt_bwd.at[pl.ds(voff, V_SLICE)], zero_buf)
    pltpu.sync_copy(zero_buf, part_hbm.at[pl.ds((c * 2 + 1) * V_PAD + voff, V_SLICE)])


@functools.partial(
    pl.kernel,
    out_type=(jax.ShapeDtypeStruct((EDGES,), jnp.float32),
              jax.ShapeDtypeStruct((EDGES,), jnp.float32)),
    mesh=_MESH,
    scratch_types=(
        pltpu.VMEM_SHARED((V_PAD,), jnp.float32),
        pltpu.VMEM_SHARED((V_PAD,), jnp.float32),
        pltpu.VMEM((CHUNK,), jnp.int32),
        pltpu.VMEM((CHUNK,), jnp.float32),
        pltpu.VMEM((V_SLICE,), jnp.float32),
        pltpu.VMEM((V_SLICE,), jnp.float32),
        pltpu.SemaphoreType.DMA,
    ),
)
def _gather_values(recv_hbm, send_hbm, part_hbm, fwd_hbm, bwd_hbm,
                   rec_fwd, rec_bwd, idx_buf, val_buf, pa_buf, pb_buf, sem):
    c = lax.axis_index("c")
    s = lax.axis_index("s")
    voff = s * V_SLICE

    def _recip_one(which, rec):
        pltpu.sync_copy(part_hbm.at[pl.ds(which * V_PAD + voff, V_SLICE)], pa_buf)
        pltpu.sync_copy(part_hbm.at[pl.ds((2 + which) * V_PAD + voff, V_SLICE)], pb_buf)

        def _r(i, _):
            tot = pa_buf[pl.ds(i * _LANES, _LANES)] + pb_buf[pl.ds(i * _LANES, _LANES)]
            pa_buf[pl.ds(i * _LANES, _LANES)] = 1.0 / tot
            return 0
        lax.fori_loop(0, V_SLICE // _LANES, _r, 0)
        pltpu.sync_copy(pa_buf, rec.at[pl.ds(voff, V_SLICE)])

    _recip_one(0, rec_fwd)
    _recip_one(1, rec_bwd)
    plsc.subcore_barrier()

    gbase = (s * NUM_CORES + c) * EDGES_PER_TILE

    def _gath(k, _):
        base = gbase + k * CHUNK
        pltpu.sync_copy(recv_hbm.at[pl.ds(base, CHUNK)], idx_buf)
        pltpu.async_copy(rec_fwd.at[idx_buf], val_buf, sem).wait()
        pltpu.sync_copy(val_buf, fwd_hbm.at[pl.ds(base, CHUNK)])
        pltpu.sync_copy(send_hbm.at[pl.ds(base, CHUNK)], idx_buf)
        pltpu.async_copy(rec_bwd.at[idx_buf], val_buf, sem).wait()
        pltpu.sync_copy(val_buf, bwd_hbm.at[pl.ds(base, CHUNK)])
        return 0
    lax.fori_loop(0, TILE_CHUNKS, _gath, 0)


def kernel(X):
    partials, receivers, senders = _count_partials(X)
    fwd_values, bwd_values = _gather_values(receivers, senders, partials)
    message_indices = jnp.arange(EDGES, dtype=X.dtype)
    return (receivers, message_indices, fwd_values,
            senders, message_indices, bwd_values)


# speedup vs baseline: 28.4823x; 28.4823x over previous
import functools

import jax
import jax.numpy as jnp
from jax import lax
from jax.experimental import pallas as pl
from jax.experimental.pallas import tpu as pltpu
from jax.experimental.pallas import tpu_sc as plsc

VERTEXES = 100000
EDGES = 1600000

NUM_CORES = 2
NUM_SUBCORES = 16
NUM_TILES = NUM_CORES * NUM_SUBCORES

V_SLICE = 6256
V_PAD = V_SLICE * NUM_SUBCORES

CHUNK = 10000
EDGES_PER_TILE = EDGES // NUM_TILES
TILE_CHUNKS = EDGES_PER_TILE // CHUNK

_LANES = 16

_MESH = plsc.VectorSubcoreMesh(core_axis_name="c", subcore_axis_name="s",
                               num_cores=NUM_CORES, num_subcores=NUM_SUBCORES)


@functools.partial(
    pl.kernel,
    out_type=jax.ShapeDtypeStruct((NUM_CORES * 2 * V_PAD,), jnp.float32),
    mesh=_MESH,
    scratch_types=(
        pltpu.VMEM_SHARED((V_PAD,), jnp.float32),
        pltpu.VMEM_SHARED((V_PAD,), jnp.float32),
        pltpu.VMEM((CHUNK,), jnp.int32),
        pltpu.VMEM((CHUNK,), jnp.float32),
        pltpu.VMEM((V_SLICE,), jnp.float32),
    ),
)
def _count_partials(recv_hbm, send_hbm, part_hbm, cnt_fwd, cnt_bwd,
                    idx_buf, ones_buf, zero_buf):
    c = lax.axis_index("c")
    s = lax.axis_index("s")

    def _fill(i, _):
        ones_buf[pl.ds(i * _LANES, _LANES)] = jnp.full((_LANES,), 1.0, jnp.float32)
        return 0
    lax.fori_loop(0, CHUNK // _LANES, _fill, 0)

    def _zero(i, _):
        zero_buf[pl.ds(i * _LANES, _LANES)] = jnp.zeros((_LANES,), jnp.float32)
        return 0
    lax.fori_loop(0, V_SLICE // _LANES, _zero, 0)
    voff = s * V_SLICE
    pltpu.sync_copy(zero_buf, cnt_fwd.at[pl.ds(voff, V_SLICE)])
    pltpu.sync_copy(zero_buf, cnt_bwd.at[pl.ds(voff, V_SLICE)])
    plsc.subcore_barrier()

    gbase = (s * NUM_CORES + c) * EDGES_PER_TILE

    def _hist(k, _):
        base = gbase + k * CHUNK
        pltpu.sync_copy(recv_hbm.at[pl.ds(base, CHUNK)], idx_buf)
        pltpu.sync_copy(ones_buf, cnt_fwd.at[idx_buf], add=True)
        pltpu.sync_copy(send_hbm.at[pl.ds(base, CHUNK)], idx_buf)
        pltpu.sync_copy(ones_buf, cnt_bwd.at[idx_buf], add=True)
        return 0
    lax.fori_loop(0, TILE_CHUNKS, _hist, 0)
    plsc.subcore_barrier()

    pltpu.sync_copy(cnt_fwd.at[pl.ds(voff, V_SLICE)], zero_buf)
    pltpu.sync_copy(zero_buf, part_hbm.at[pl.ds(c * 2 * V_PAD + voff, V_SLICE)])
    pltpu.sync_copy(cnt_bwd.at[pl.ds(voff, V_SLICE)], zero_buf)
    pltpu.sync_copy(zero_buf, part_hbm.at[pl.ds((c * 2 + 1) * V_PAD + voff, V_SLICE)])


@functools.partial(
    pl.kernel,
    out_type=(jax.ShapeDtypeStruct((EDGES,), jnp.float32),
              jax.ShapeDtypeStruct((EDGES,), jnp.float32)),
    mesh=_MESH,
    scratch_types=(
        pltpu.VMEM_SHARED((V_PAD,), jnp.float32),
        pltpu.VMEM_SHARED((V_PAD,), jnp.float32),
        pltpu.VMEM((CHUNK,), jnp.int32),
        pltpu.VMEM((CHUNK,), jnp.float32),
        pltpu.VMEM((V_SLICE,), jnp.float32),
        pltpu.VMEM((V_SLICE,), jnp.float32),
        pltpu.SemaphoreType.DMA,
    ),
)
def _gather_values(recv_hbm, send_hbm, part_hbm, fwd_hbm, bwd_hbm,
                   rec_fwd, rec_bwd, idx_buf, val_buf, pa_buf, pb_buf, sem):
    c = lax.axis_index("c")
    s = lax.axis_index("s")
    voff = s * V_SLICE

    def _recip_one(which, rec):
        pltpu.sync_copy(part_hbm.at[pl.ds(which * V_PAD + voff, V_SLICE)], pa_buf)
        pltpu.sync_copy(part_hbm.at[pl.ds((2 + which) * V_PAD + voff, V_SLICE)], pb_buf)

        def _r(i, _):
            tot = pa_buf[pl.ds(i * _LANES, _LANES)] + pb_buf[pl.ds(i * _LANES, _LANES)]
            pa_buf[pl.ds(i * _LANES, _LANES)] = 1.0 / tot
            return 0
        lax.fori_loop(0, V_SLICE // _LANES, _r, 0)
        pltpu.sync_copy(pa_buf, rec.at[pl.ds(voff, V_SLICE)])

    _recip_one(0, rec_fwd)
    _recip_one(1, rec_bwd)
    plsc.subcore_barrier()

    gbase = (s * NUM_CORES + c) * EDGES_PER_TILE

    def _gath(k, _):
        base = gbase + k * CHUNK
        pltpu.sync_copy(recv_hbm.at[pl.ds(base, CHUNK)], idx_buf)
        pltpu.async_copy(rec_fwd.at[idx_buf], val_buf, sem).wait()
        pltpu.sync_copy(val_buf, fwd_hbm.at[pl.ds(base, CHUNK)])
        pltpu.sync_copy(send_hbm.at[pl.ds(base, CHUNK)], idx_buf)
        pltpu.async_copy(rec_bwd.at[idx_buf], val_buf, sem).wait()
        pltpu.sync_copy(val_buf, bwd_hbm.at[pl.ds(base, CHUNK)])
        return 0
    lax.fori_loop(0, TILE_CHUNKS, _gath, 0)


def kernel(X):
    receivers = X[:, 2]
    senders = X[:, 0]
    partials = _count_partials(receivers, senders)
    fwd_values, bwd_values = _gather_values(receivers, senders, partials)
    message_indices = jnp.arange(EDGES, dtype=X.dtype)
    return (receivers, message_indices, fwd_values,
            senders, message_indices, bwd_values)
